# baseline (speedup 1.0000x reference)
import functools

import jax
import jax.numpy as jnp
from jax import lax
from jax.experimental import pallas as pl
from jax.experimental.pallas import tpu as pltpu
from jax.experimental.pallas import tpu_sc as plsc

N_NODES = 50000
N_EDGES = 800000
HID = 128
LANES = 16

CB = 4096
NCHUNK = 196
E_OUT = NCHUNK * CB
LAST_BASE = N_EDGES - CB
NBIG = 4
CN = 4096


def _sc_gather(px, py, eflat):
    mesh = plsc.VectorSubcoreMesh(core_axis_name="c", subcore_axis_name="s")

    @functools.partial(
        pl.kernel,
        out_type=(
            jax.ShapeDtypeStruct((E_OUT,), jnp.float32),
            jax.ShapeDtypeStruct((E_OUT,), jnp.float32),
        ),
        mesh=mesh,
        scratch_types=[
            pltpu.VMEM((N_NODES,), jnp.float32),
            pltpu.VMEM((N_NODES,), jnp.float32),
            pltpu.VMEM((2 * CB,), jnp.int32),
            pltpu.VMEM((CB,), jnp.float32),
            pltpu.VMEM((CB,), jnp.float32),
        ],
        compiler_params=pltpu.CompilerParams(needs_layout_passes=False),
    )
    def gather_kernel(px_hbm, py_hbm, e_hbm, dx_hbm, dy_hbm,
                      px_v, py_v, ev, dxb, dyb):
        wid = lax.axis_index("s") * 2 + lax.axis_index("c")
        pltpu.sync_copy(px_hbm, px_v)
        pltpu.sync_copy(py_hbm, py_v)
        nb_w = jnp.where(wid < NBIG, 7, 6)
        b0 = jnp.where(wid < NBIG, 7 * wid, 7 * NBIG + 6 * (wid - NBIG))
        lane2 = lax.iota(jnp.int32, LANES) * 2

        def chunk_body(ch, _):
            base = jnp.minimum((b0 + ch) * CB, LAST_BASE)
            pltpu.sync_copy(e_hbm.at[pl.ds(2 * base, 2 * CB)], ev)

            @plsc.parallel_loop(0, CB // LANES, unroll=8)
            def vec_body(k):
                o = k * LANES
                i2 = 2 * o + lane2
                si = plsc.load_gather(ev, [i2])
                ri = plsc.load_gather(ev, [i2 + 1])
                dxb[pl.ds(o, LANES)] = (plsc.load_gather(px_v, [si])
                                        - plsc.load_gather(px_v, [ri]))
                dyb[pl.ds(o, LANES)] = (plsc.load_gather(py_v, [si])
                                        - plsc.load_gather(py_v, [ri]))
            pltpu.sync_copy(dxb, dx_hbm.at[pl.ds(base, CB)])
            pltpu.sync_copy(dyb, dy_hbm.at[pl.ds(base, CB)])
            return 0

        lax.fori_loop(0, nb_w, chunk_body, 0)

    return gather_kernel(px, py, eflat)


def _edge_mlp_body(dxl_ref, dyl_ref, gmat_ref, w2_ref, nb2_ref,
                   w3_ref, b3_ref, g_ref, beta_ref, onesd_ref, out_ref):
    bf16 = jnp.bfloat16
    nj = CB // HID
    dxt = jnp.transpose(dxl_ref[...], (1, 0))
    dyt = jnp.transpose(dyl_ref[...], (1, 0))
    nrmt = jnp.sqrt(dxt * dxt + dyt * dyt)
    S = jnp.concatenate(
        [dxt, dyt, nrmt, jnp.ones((HID, nj), jnp.float32)],
        axis=1).astype(bf16)
    H = jnp.dot(S, gmat_ref[...],
                preferred_element_type=jnp.float32)
    H = jnp.maximum(H, 0.0).astype(bf16)
    h = jnp.concatenate([H[:, j * HID:(j + 1) * HID] for j in range(nj)],
                        axis=0)
    h = jnp.dot(h, w2_ref[...], preferred_element_type=jnp.float32)
    h = jnp.maximum(h, nb2_ref[...]).astype(bf16)
    d = jnp.dot(h, w3_ref[...],
                preferred_element_type=jnp.float32) + b3_ref[...]
    db = d.astype(bf16)
    var = jnp.dot(db * db, onesd_ref[...],
                  preferred_element_type=jnp.float32)
    out_ref[...] = g_ref[...] * (d * lax.rsqrt(var + 1e-5)) + beta_ref[...]


def _node_mlp_body(v_ref, w1_ref, b1_ref, w2_ref, nb2_ref,
                   w3_ref, b3_ref, g_ref, beta_ref, onesd_ref, out_ref):
    bf16 = jnp.bfloat16
    h = (jnp.dot(v_ref[...], w1_ref[...], preferred_element_type=jnp.float32)
         + b1_ref[...])
    h = jnp.maximum(h, 0.0).astype(bf16)
    h = jnp.dot(h, w2_ref[...], preferred_element_type=jnp.float32)
    h = jnp.maximum(h, nb2_ref[...]).astype(bf16)
    d = jnp.dot(h, w3_ref[...],
                preferred_element_type=jnp.float32) + b3_ref[...]
    db = d.astype(bf16)
    var = jnp.dot(db * db, onesd_ref[...],
                  preferred_element_type=jnp.float32)
    out_ref[...] = g_ref[...] * (d * lax.rsqrt(var + 1e-5)) + beta_ref[...]


def _const_spec(shape):
    return pl.BlockSpec(shape, lambda i: tuple(0 for _ in shape))


def kernel(mesh_pos, edges, node_type, state_hat, parameters,
           fv_W1, fv_b1, fv_W2, fv_b2, fv_W3, fv_b3, fv_ln_g, fv_ln_b,
           fe_W1, fe_b1, fe_W2, fe_b2, fe_W3, fe_b3, fe_ln_g, fe_ln_b):
    f32 = jnp.float32
    bf16 = jnp.bfloat16
    inv = f32(1.0 / (1.0 + 1e-8))
    row = lambda v: v.reshape(1, HID)
    onesd = jnp.full((HID, HID), 1.0 / HID, bf16)

    px = mesh_pos[0, :, 0]
    py = mesh_pos[0, :, 1]
    eflat = edges.reshape(2 * N_EDGES)
    fe_W3c = (fe_W3 - jnp.mean(fe_W3, axis=1, keepdims=True)).astype(bf16)
    fe_b3c = fe_b3 - jnp.mean(fe_b3)
    fv_W3c = (fv_W3 - jnp.mean(fv_W3, axis=1, keepdims=True)).astype(bf16)
    fv_b3c = fv_b3 - jnp.mean(fv_b3)
    fe_b3p = fe_b3c + (fe_b2.astype(bf16) @ fe_W3c).astype(f32)
    fv_b3p = fv_b3c + (fv_b2.astype(bf16) @ fv_W3c).astype(f32)

    nj = CB // HID
    W1e = jnp.concatenate([fe_W1 * inv, row(fe_b1)], axis=0)
    gmat = (jnp.eye(nj, dtype=f32)[None, :, :, None]
            * W1e[:, None, None, :]).reshape(4 * nj, nj * HID).astype(bf16)

    dxl, dyl = _sc_gather(px, py, eflat)
    dxl = dxl.reshape(E_OUT // HID, HID)
    dyl = dyl.reshape(E_OUT // HID, HID)

    grid_e = (N_EDGES + CB - 1) // CB
    eout = pl.pallas_call(
        _edge_mlp_body,
        grid=(grid_e,),
        in_specs=[
            pl.BlockSpec((CB // HID, HID), lambda i: (i, 0)),
            pl.BlockSpec((CB // HID, HID), lambda i: (i, 0)),
            _const_spec((HID, CB)),
            _const_spec((HID, HID)),
            _const_spec((1, HID)),
            _const_spec((HID, HID)),
            _const_spec((1, HID)),
            _const_spec((1, HID)),
            _const_spec((1, HID)),
            _const_spec((HID, HID)),
        ],
        out_specs=pl.BlockSpec((CB, HID), lambda i: (i, 0)),
        out_shape=jax.ShapeDtypeStruct((N_EDGES, HID), f32),
    )(dxl, dyl, gmat, fe_W2.astype(bf16), -row(fe_b2),
      fe_W3c, row(fe_b3p), row(fe_ln_g), row(fe_ln_b), onesd)

    V = jnp.concatenate([state_hat[0], node_type[0], parameters[0]], axis=-1)
    grid_n = (N_NODES + CN - 1) // CN
    vout = pl.pallas_call(
        _node_mlp_body,
        grid=(grid_n,),
        in_specs=[
            pl.BlockSpec((CN, 16), lambda i: (i, 0)),
            _const_spec((16, HID)),
            _const_spec((1, HID)),
            _const_spec((HID, HID)),
            _const_spec((1, HID)),
            _const_spec((HID, HID)),
            _const_spec((1, HID)),
            _const_spec((1, HID)),
            _const_spec((1, HID)),
            _const_spec((HID, HID)),
        ],
        out_specs=pl.BlockSpec((CN, HID), lambda i: (i, 0)),
        out_shape=jax.ShapeDtypeStruct((N_NODES, HID), f32),
    )(V, fv_W1 * inv, row(fv_b1), fv_W2.astype(bf16), -row(fv_b2),
      fv_W3c, row(fv_b3p), row(fv_ln_g), row(fv_ln_b), onesd)

    return (vout.reshape(1, N_NODES, HID), eout.reshape(1, N_EDGES, HID))


# speedup vs baseline: 3.7806x; 3.7806x over previous
import functools

import jax
import jax.numpy as jnp
from jax import lax
from jax.experimental import pallas as pl
from jax.experimental.pallas import tpu as pltpu
from jax.experimental.pallas import tpu_sc as plsc

N_NODES = 50000
N_EDGES = 800000
HID = 128
LANES = 16

CB = 4096
CBT = 8192
E_PAD = 819200
NBLK = E_PAD // CB
NBIG = 8
CN = 4096


def _sc_gather(px, py, s_idx, r_idx):
    mesh = plsc.VectorSubcoreMesh(core_axis_name="c", subcore_axis_name="s")

    @functools.partial(
        pl.kernel,
        out_type=(
            jax.ShapeDtypeStruct((E_PAD,), jnp.float32),
            jax.ShapeDtypeStruct((E_PAD,), jnp.float32),
        ),
        mesh=mesh,
        scratch_types=[
            pltpu.VMEM((N_NODES,), jnp.float32),
            pltpu.VMEM((N_NODES,), jnp.float32),
            pltpu.VMEM((CB,), jnp.int32),
            pltpu.VMEM((CB,), jnp.int32),
            pltpu.VMEM((CB,), jnp.float32),
            pltpu.VMEM((CB,), jnp.float32),
        ],
        compiler_params=pltpu.CompilerParams(needs_layout_passes=False),
    )
    def gather_kernel(px_hbm, py_hbm, s_hbm, r_hbm, dx_hbm, dy_hbm,
                      px_v, py_v, sv, rv, dxb, dyb):
        wid = lax.axis_index("s") * 2 + lax.axis_index("c")
        pltpu.sync_copy(px_hbm, px_v)
        pltpu.sync_copy(py_hbm, py_v)
        nb_w = jnp.where(wid < NBIG, 7, 6)
        b0 = jnp.where(wid < NBIG, 7 * wid, 7 * NBIG + 6 * (wid - NBIG))

        def chunk_body(ch, _):
            base = (b0 + ch) * CB
            pltpu.sync_copy(s_hbm.at[pl.ds(base, CB)], sv)
            pltpu.sync_copy(r_hbm.at[pl.ds(base, CB)], rv)

            @plsc.parallel_loop(0, CB // LANES, unroll=8)
            def vec_body(k):
                o = k * LANES
                si = sv[pl.ds(o, LANES)]
                ri = rv[pl.ds(o, LANES)]
                dxb[pl.ds(o, LANES)] = (plsc.load_gather(px_v, [si])
                                        - plsc.load_gather(px_v, [ri]))
                dyb[pl.ds(o, LANES)] = (plsc.load_gather(py_v, [si])
                                        - plsc.load_gather(py_v, [ri]))
            pltpu.sync_copy(dxb, dx_hbm.at[pl.ds(base, CB)])
            pltpu.sync_copy(dyb, dy_hbm.at[pl.ds(base, CB)])
            return 0

        lax.fori_loop(0, nb_w, chunk_body, 0)

    return gather_kernel(px, py, s_idx, r_idx)


def _edge_mlp_body(dxl_ref, dyl_ref, gmat_ref, w2_ref, nb2_ref,
                   w3_ref, b3_ref, g_ref, beta_ref, onesd_ref, out_ref):
    bf16 = jnp.bfloat16
    nj = 4096 // HID
    parts = []
    for half in range(CBT // 4096):
        r0 = half * (4096 // HID)
        dxt = jnp.transpose(dxl_ref[r0:r0 + nj, :], (1, 0))
        dyt = jnp.transpose(dyl_ref[r0:r0 + nj, :], (1, 0))
        nrmt = jnp.sqrt(dxt * dxt + dyt * dyt)
        S = jnp.concatenate(
            [dxt, dyt, nrmt, jnp.ones((HID, nj), jnp.float32)],
            axis=1).astype(bf16)
        H = jnp.dot(S, gmat_ref[...],
                    preferred_element_type=jnp.float32)
        H = jnp.maximum(H, 0.0).astype(bf16)
        parts.extend(H[:, j * HID:(j + 1) * HID] for j in range(nj))
    h = jnp.concatenate(parts, axis=0)
    h = jnp.dot(h, w2_ref[...], preferred_element_type=jnp.float32)
    h = jnp.maximum(h, nb2_ref[...]).astype(bf16)
    d = jnp.dot(h, w3_ref[...],
                preferred_element_type=jnp.float32) + b3_ref[...]
    db = d.astype(bf16)
    var = jnp.dot(db * db, onesd_ref[...],
                  preferred_element_type=jnp.float32)
    out_ref[...] = g_ref[...] * (d * lax.rsqrt(var + 1e-5)) + beta_ref[...]


def _node_mlp_body(v_ref, w1_ref, b1_ref, w2_ref, nb2_ref,
                   w3_ref, b3_ref, g_ref, beta_ref, onesd_ref, out_ref):
    bf16 = jnp.bfloat16
    h = (jnp.dot(v_ref[...], w1_ref[...], preferred_element_type=jnp.float32)
         + b1_ref[...])
    h = jnp.maximum(h, 0.0).astype(bf16)
    h = jnp.dot(h, w2_ref[...], preferred_element_type=jnp.float32)
    h = jnp.maximum(h, nb2_ref[...]).astype(bf16)
    d = jnp.dot(h, w3_ref[...],
                preferred_element_type=jnp.float32) + b3_ref[...]
    db = d.astype(bf16)
    var = jnp.dot(db * db, onesd_ref[...],
                  preferred_element_type=jnp.float32)
    out_ref[...] = g_ref[...] * (d * lax.rsqrt(var + 1e-5)) + beta_ref[...]


def _const_spec(shape):
    return pl.BlockSpec(shape, lambda i: tuple(0 for _ in shape))


def kernel(mesh_pos, edges, node_type, state_hat, parameters,
           fv_W1, fv_b1, fv_W2, fv_b2, fv_W3, fv_b3, fv_ln_g, fv_ln_b,
           fe_W1, fe_b1, fe_W2, fe_b2, fe_W3, fe_b3, fe_ln_g, fe_ln_b):
    f32 = jnp.float32
    bf16 = jnp.bfloat16
    inv = f32(1.0 / (1.0 + 1e-8))
    row = lambda v: v.reshape(1, HID)
    onesd = jnp.full((HID, HID), 1.0 / HID, bf16)

    px = mesh_pos[0, :, 0]
    py = mesh_pos[0, :, 1]
    s_idx = jnp.pad(edges[0, :, 0], (0, E_PAD - N_EDGES))
    r_idx = jnp.pad(edges[0, :, 1], (0, E_PAD - N_EDGES))
    fe_W3c = (fe_W3 - jnp.mean(fe_W3, axis=1, keepdims=True)).astype(bf16)
    fe_b3c = fe_b3 - jnp.mean(fe_b3)
    fv_W3c = (fv_W3 - jnp.mean(fv_W3, axis=1, keepdims=True)).astype(bf16)
    fv_b3c = fv_b3 - jnp.mean(fv_b3)
    fe_b3p = fe_b3c + (fe_b2.astype(bf16) @ fe_W3c).astype(f32)
    fv_b3p = fv_b3c + (fv_b2.astype(bf16) @ fv_W3c).astype(f32)

    nj = 4096 // HID
    W1e = jnp.concatenate([fe_W1 * inv, row(fe_b1)], axis=0)
    gmat = (jnp.eye(nj, dtype=f32)[None, :, :, None]
            * W1e[:, None, None, :]).reshape(4 * nj, nj * HID).astype(bf16)

    dxl, dyl = _sc_gather(px, py, s_idx, r_idx)
    dxl = dxl.reshape(E_PAD // HID, HID)
    dyl = dyl.reshape(E_PAD // HID, HID)

    grid_e = (N_EDGES + CBT - 1) // CBT
    eout = pl.pallas_call(
        _edge_mlp_body,
        grid=(grid_e,),
        in_specs=[
            pl.BlockSpec((CBT // HID, HID), lambda i: (i, 0)),
            pl.BlockSpec((CBT // HID, HID), lambda i: (i, 0)),
            _const_spec((HID, 4096)),
            _const_spec((HID, HID)),
            _const_spec((1, HID)),
            _const_spec((HID, HID)),
            _const_spec((1, HID)),
            _const_spec((1, HID)),
            _const_spec((1, HID)),
            _const_spec((HID, HID)),
        ],
        out_specs=pl.BlockSpec((CBT, HID), lambda i: (i, 0)),
        out_shape=jax.ShapeDtypeStruct((N_EDGES, HID), f32),
    )(dxl, dyl, gmat, fe_W2.astype(bf16), -row(fe_b2),
      fe_W3c, row(fe_b3p), row(fe_ln_g), row(fe_ln_b), onesd)

    V = jnp.concatenate([state_hat[0], node_type[0], parameters[0]], axis=-1)
    grid_n = (N_NODES + CN - 1) // CN
    vout = pl.pallas_call(
        _node_mlp_body,
        grid=(grid_n,),
        in_specs=[
            pl.BlockSpec((CN, 16), lambda i: (i, 0)),
            _const_spec((16, HID)),
            _const_spec((1, HID)),
            _const_spec((HID, HID)),
            _const_spec((1, HID)),
            _const_spec((HID, HID)),
            _const_spec((1, HID)),
            _const_spec((1, HID)),
            _const_spec((1, HID)),
            _const_spec((HID, HID)),
        ],
        out_specs=pl.BlockSpec((CN, HID), lambda i: (i, 0)),
        out_shape=jax.ShapeDtypeStruct((N_NODES, HID), f32),
    )(V, fv_W1 * inv, row(fv_b1), fv_W2.astype(bf16), -row(fv_b2),
      fv_W3c, row(fv_b3p), row(fv_ln_g), row(fv_ln_b), onesd)

    return (vout.reshape(1, N_NODES, HID), eout.reshape(1, N_EDGES, HID))


# speedup vs baseline: 4.0429x; 1.0694x over previous
import functools

import jax
import jax.numpy as jnp
from jax import lax
from jax.experimental import pallas as pl
from jax.experimental.pallas import tpu as pltpu
from jax.experimental.pallas import tpu_sc as plsc

N_NODES = 50000
N_EDGES = 800000
HID = 128
LANES = 16

CB = 4096
CBT = 16384
E_PAD = 819200
NBLK = E_PAD // CB
NBIG = 8
CN = 4096


def _sc_gather(px, py, s_idx, r_idx):
    mesh = plsc.VectorSubcoreMesh(core_axis_name="c", subcore_axis_name="s")

    @functools.partial(
        pl.kernel,
        out_type=(
            jax.ShapeDtypeStruct((E_PAD,), jnp.float32),
            jax.ShapeDtypeStruct((E_PAD,), jnp.float32),
        ),
        mesh=mesh,
        scratch_types=[
            pltpu.VMEM((N_NODES,), jnp.float32),
            pltpu.VMEM((N_NODES,), jnp.float32),
            pltpu.VMEM((CB,), jnp.int32),
            pltpu.VMEM((CB,), jnp.int32),
            pltpu.VMEM((CB,), jnp.float32),
            pltpu.VMEM((CB,), jnp.float32),
        ],
        compiler_params=pltpu.CompilerParams(needs_layout_passes=False),
    )
    def gather_kernel(px_hbm, py_hbm, s_hbm, r_hbm, dx_hbm, dy_hbm,
                      px_v, py_v, sv, rv, dxb, dyb):
        wid = lax.axis_index("s") * 2 + lax.axis_index("c")
        pltpu.sync_copy(px_hbm, px_v)
        pltpu.sync_copy(py_hbm, py_v)
        nb_w = jnp.where(wid < NBIG, 7, 6)
        b0 = jnp.where(wid < NBIG, 7 * wid, 7 * NBIG + 6 * (wid - NBIG))

        def chunk_body(ch, _):
            base = (b0 + ch) * CB
            pltpu.sync_copy(s_hbm.at[pl.ds(base, CB)], sv)
            pltpu.sync_copy(r_hbm.at[pl.ds(base, CB)], rv)

            @plsc.parallel_loop(0, CB // LANES, unroll=8)
            def vec_body(k):
                o = k * LANES
                si = sv[pl.ds(o, LANES)]
                ri = rv[pl.ds(o, LANES)]
                dxb[pl.ds(o, LANES)] = (plsc.load_gather(px_v, [si])
                                        - plsc.load_gather(px_v, [ri]))
                dyb[pl.ds(o, LANES)] = (plsc.load_gather(py_v, [si])
                                        - plsc.load_gather(py_v, [ri]))
            pltpu.sync_copy(dxb, dx_hbm.at[pl.ds(base, CB)])
            pltpu.sync_copy(dyb, dy_hbm.at[pl.ds(base, CB)])
            return 0

        lax.fori_loop(0, nb_w, chunk_body, 0)

    return gather_kernel(px, py, s_idx, r_idx)


def _edge_mlp_body(dxl_ref, dyl_ref, gmat_ref, w2_ref, nb2_ref,
                   w3_ref, b3_ref, g_ref, beta_ref, onesd_ref, out_ref):
    bf16 = jnp.bfloat16
    nj = 4096 // HID
    parts = []
    for half in range(CBT // 4096):
        r0 = half * (4096 // HID)
        dxt = jnp.transpose(dxl_ref[r0:r0 + nj, :], (1, 0))
        dyt = jnp.transpose(dyl_ref[r0:r0 + nj, :], (1, 0))
        nrmt = jnp.sqrt(dxt * dxt + dyt * dyt)
        S = jnp.concatenate(
            [dxt, dyt, nrmt, jnp.ones((HID, nj), jnp.float32)],
            axis=1).astype(bf16)
        H = jnp.dot(S, gmat_ref[...],
                    preferred_element_type=jnp.float32)
        H = jnp.maximum(H, 0.0).astype(bf16)
        parts.extend(H[:, j * HID:(j + 1) * HID] for j in range(nj))
    h = jnp.concatenate(parts, axis=0)
    h = jnp.dot(h, w2_ref[...], preferred_element_type=jnp.float32)
    h = jnp.maximum(h, nb2_ref[...]).astype(bf16)
    d = jnp.dot(h, w3_ref[...],
                preferred_element_type=jnp.float32) + b3_ref[...]
    db = d.astype(bf16)
    var = jnp.dot(db * db, onesd_ref[...],
                  preferred_element_type=jnp.float32)
    out_ref[...] = g_ref[...] * (d * lax.rsqrt(var + 1e-5)) + beta_ref[...]


def _node_mlp_body(v_ref, w1_ref, b1_ref, w2_ref, nb2_ref,
                   w3_ref, b3_ref, g_ref, beta_ref, onesd_ref, out_ref):
    bf16 = jnp.bfloat16
    h = (jnp.dot(v_ref[...], w1_ref[...], preferred_element_type=jnp.float32)
         + b1_ref[...])
    h = jnp.maximum(h, 0.0).astype(bf16)
    h = jnp.dot(h, w2_ref[...], preferred_element_type=jnp.float32)
    h = jnp.maximum(h, nb2_ref[...]).astype(bf16)
    d = jnp.dot(h, w3_ref[...],
                preferred_element_type=jnp.float32) + b3_ref[...]
    db = d.astype(bf16)
    var = jnp.dot(db * db, onesd_ref[...],
                  preferred_element_type=jnp.float32)
    out_ref[...] = g_ref[...] * (d * lax.rsqrt(var + 1e-5)) + beta_ref[...]


def _const_spec(shape):
    return pl.BlockSpec(shape, lambda i: tuple(0 for _ in shape))


def kernel(mesh_pos, edges, node_type, state_hat, parameters,
           fv_W1, fv_b1, fv_W2, fv_b2, fv_W3, fv_b3, fv_ln_g, fv_ln_b,
           fe_W1, fe_b1, fe_W2, fe_b2, fe_W3, fe_b3, fe_ln_g, fe_ln_b):
    f32 = jnp.float32
    bf16 = jnp.bfloat16
    inv = f32(1.0 / (1.0 + 1e-8))
    row = lambda v: v.reshape(1, HID)
    onesd = jnp.full((HID, HID), 1.0 / HID, bf16)

    px = mesh_pos[0, :, 0]
    py = mesh_pos[0, :, 1]
    s_idx = jnp.pad(edges[0, :, 0], (0, E_PAD - N_EDGES))
    r_idx = jnp.pad(edges[0, :, 1], (0, E_PAD - N_EDGES))
    fe_W3c = (fe_W3 - jnp.mean(fe_W3, axis=1, keepdims=True)).astype(bf16)
    fe_b3c = fe_b3 - jnp.mean(fe_b3)
    fv_W3c = (fv_W3 - jnp.mean(fv_W3, axis=1, keepdims=True)).astype(bf16)
    fv_b3c = fv_b3 - jnp.mean(fv_b3)
    fe_b3p = fe_b3c + (fe_b2.astype(bf16) @ fe_W3c).astype(f32)
    fv_b3p = fv_b3c + (fv_b2.astype(bf16) @ fv_W3c).astype(f32)

    nj = 4096 // HID
    W1e = jnp.concatenate([fe_W1 * inv, row(fe_b1)], axis=0)
    gmat = (jnp.eye(nj, dtype=f32)[None, :, :, None]
            * W1e[:, None, None, :]).reshape(4 * nj, nj * HID).astype(bf16)

    dxl, dyl = _sc_gather(px, py, s_idx, r_idx)
    dxl = dxl.reshape(E_PAD // HID, HID)
    dyl = dyl.reshape(E_PAD // HID, HID)

    grid_e = (N_EDGES + CBT - 1) // CBT
    eout = pl.pallas_call(
        _edge_mlp_body,
        grid=(grid_e,),
        in_specs=[
            pl.BlockSpec((CBT // HID, HID), lambda i: (i, 0)),
            pl.BlockSpec((CBT // HID, HID), lambda i: (i, 0)),
            _const_spec((HID, 4096)),
            _const_spec((HID, HID)),
            _const_spec((1, HID)),
            _const_spec((HID, HID)),
            _const_spec((1, HID)),
            _const_spec((1, HID)),
            _const_spec((1, HID)),
            _const_spec((HID, HID)),
        ],
        out_specs=pl.BlockSpec((CBT, HID), lambda i: (i, 0)),
        out_shape=jax.ShapeDtypeStruct((N_EDGES, HID), f32),
    )(dxl, dyl, gmat, fe_W2.astype(bf16), -row(fe_b2),
      fe_W3c, row(fe_b3p), row(fe_ln_g), row(fe_ln_b), onesd)

    V = jnp.concatenate([state_hat[0], node_type[0], parameters[0]], axis=-1)
    grid_n = (N_NODES + CN - 1) // CN
    vout = pl.pallas_call(
        _node_mlp_body,
        grid=(grid_n,),
        in_specs=[
            pl.BlockSpec((CN, 16), lambda i: (i, 0)),
            _const_spec((16, HID)),
            _const_spec((1, HID)),
            _const_spec((HID, HID)),
            _const_spec((1, HID)),
            _const_spec((HID, HID)),
            _const_spec((1, HID)),
            _const_spec((1, HID)),
            _const_spec((1, HID)),
            _const_spec((HID, HID)),
        ],
        out_specs=pl.BlockSpec((CN, HID), lambda i: (i, 0)),
        out_shape=jax.ShapeDtypeStruct((N_NODES, HID), f32),
    )(V, fv_W1 * inv, row(fv_b1), fv_W2.astype(bf16), -row(fv_b2),
      fv_W3c, row(fv_b3p), row(fv_ln_g), row(fv_ln_b), onesd)

    return (vout.reshape(1, N_NODES, HID), eout.reshape(1, N_EDGES, HID))
